# baseline (speedup 1.0000x reference)
import functools

import jax
import jax.numpy as jnp
from jax import lax
from jax.experimental import pallas as pl
from jax.experimental.pallas import tpu as pltpu
from jax.experimental.pallas import tpu_sc as plsc

VOCAB = 100000
DIM = 300
BATCH = 1024
MAX_NORM = 0.15


_NC, _NS = 2, 16
_NW = _NC * _NS
_B_PER_W = BATCH // _NW
_LANES = 16


def _sc_gather(table, idx):
    mesh = plsc.VectorSubcoreMesh(core_axis_name="c", subcore_axis_name="s")

    @functools.partial(
        pl.kernel,
        mesh=mesh,
        out_type=jax.ShapeDtypeStruct((BATCH, DIM), jnp.float32),
        scratch_types=[
            pltpu.VMEM((_B_PER_W,), jnp.int32),
            pltpu.VMEM((_B_PER_W, DIM), jnp.float32),
            pltpu.SemaphoreType.DMA,
        ],
    )
    def gather_kernel(table_hbm, idx_hbm, out_hbm, idx_v, rows_v, sem):
        wid = lax.axis_index("s") * _NC + lax.axis_index("c")
        base = wid * _B_PER_W
        pltpu.sync_copy(idx_hbm.at[pl.ds(base, _B_PER_W)], idx_v)
        copies = []
        for c in range(_B_PER_W // _LANES):
            chunk = idx_v[pl.ds(c * _LANES, _LANES)]
            for k in range(_LANES):
                j = c * _LANES + k
                row = chunk[k]
                cp = pltpu.make_async_copy(
                    table_hbm.at[pl.ds(row, 1)], rows_v.at[pl.ds(j, 1)], sem
                )
                cp.start()
                copies.append(cp)
        for cp in copies:
            cp.wait()
        pltpu.sync_copy(rows_v, out_hbm.at[pl.ds(base, _B_PER_W)])

    return gather_kernel(table, idx)


TILE_V = 2048


def _mm_body(x_ref, w_ref, b_ref, out_ref, xs_ref):
    @pl.when(pl.program_id(0) == 0)
    def _():
        x = x_ref[...]
        nrm = jnp.sqrt(jnp.sum(x * x, axis=1, keepdims=True))
        scale = jnp.where(nrm > MAX_NORM, MAX_NORM / (nrm + 1e-7), 1.0)
        xs_ref[...] = (x * scale).astype(jnp.bfloat16)

    acc = lax.dot_general(
        xs_ref[...],
        w_ref[...].astype(jnp.bfloat16),
        (((1,), (1,)), ((), ())),
        preferred_element_type=jnp.float32,
    )
    out_ref[...] = acc + b_ref[...]


def _matmul(x, W, b2, interpret=False):
    n_tiles = pl.cdiv(VOCAB, TILE_V)
    return pl.pallas_call(
        _mm_body,
        grid=(n_tiles,),
        in_specs=[
            pl.BlockSpec((BATCH, DIM), lambda i: (0, 0)),
            pl.BlockSpec((TILE_V, DIM), lambda i: (i, 0)),
            pl.BlockSpec((1, TILE_V), lambda i: (0, i)),
        ],
        out_specs=pl.BlockSpec((BATCH, TILE_V), lambda i: (0, i)),
        out_shape=jax.ShapeDtypeStruct((BATCH, VOCAB), jnp.float32),
        scratch_shapes=[pltpu.VMEM((BATCH, DIM), jnp.bfloat16)],
        interpret=interpret,
    )(x, W, b2)


@jax.jit
def kernel(_inputs, target_table, W, b):
    idx = _inputs.astype(jnp.int32)
    x_raw = _sc_gather(target_table, idx)
    return _matmul(x_raw, W, b.reshape(1, VOCAB))


# speedup vs baseline: 1.0072x; 1.0072x over previous
import functools

import jax
import jax.numpy as jnp
from jax import lax
from jax.experimental import pallas as pl
from jax.experimental.pallas import tpu as pltpu
from jax.experimental.pallas import tpu_sc as plsc

VOCAB = 100000
DIM = 300
BATCH = 1024
MAX_NORM = 0.15


_NC, _NS = 2, 16
_NW = _NC * _NS
_B_PER_W = BATCH // _NW
_LANES = 16


def _sc_gather(table, idx):
    mesh = plsc.VectorSubcoreMesh(core_axis_name="c", subcore_axis_name="s")

    @functools.partial(
        pl.kernel,
        mesh=mesh,
        out_type=jax.ShapeDtypeStruct((BATCH, DIM), jnp.float32),
        scratch_types=[
            pltpu.VMEM((_B_PER_W,), jnp.int32),
            pltpu.VMEM((_B_PER_W, DIM), jnp.float32),
            pltpu.SemaphoreType.DMA,
        ],
    )
    def gather_kernel(table_hbm, idx_hbm, out_hbm, idx_v, rows_v, sem):
        wid = lax.axis_index("s") * _NC + lax.axis_index("c")
        base = wid * _B_PER_W
        pltpu.sync_copy(idx_hbm.at[pl.ds(base, _B_PER_W)], idx_v)
        copies = []
        for c in range(_B_PER_W // _LANES):
            chunk = idx_v[pl.ds(c * _LANES, _LANES)]
            for k in range(_LANES):
                j = c * _LANES + k
                row = chunk[k]
                cp = pltpu.make_async_copy(
                    table_hbm.at[pl.ds(row, 1)], rows_v.at[pl.ds(j, 1)], sem
                )
                cp.start()
                copies.append(cp)
        for cp in copies:
            cp.wait()
        pltpu.sync_copy(rows_v, out_hbm.at[pl.ds(base, _B_PER_W)])

    return gather_kernel(table, idx)


TILE_V = 4096


def _mm_body(x_ref, w_ref, b_ref, out_ref, xs_ref):
    @pl.when(pl.program_id(0) == 0)
    def _():
        x = x_ref[...]
        nrm = jnp.sqrt(jnp.sum(x * x, axis=1, keepdims=True))
        scale = jnp.where(nrm > MAX_NORM, MAX_NORM / (nrm + 1e-7), 1.0)
        xs_ref[...] = (x * scale).astype(jnp.bfloat16)

    acc = lax.dot_general(
        xs_ref[...],
        w_ref[...].astype(jnp.bfloat16),
        (((1,), (1,)), ((), ())),
        preferred_element_type=jnp.float32,
    )
    out_ref[...] = acc + b_ref[...]


def _matmul(x, W, b2, interpret=False):
    n_tiles = pl.cdiv(VOCAB, TILE_V)
    return pl.pallas_call(
        _mm_body,
        grid=(n_tiles,),
        in_specs=[
            pl.BlockSpec((BATCH, DIM), lambda i: (0, 0)),
            pl.BlockSpec((TILE_V, DIM), lambda i: (i, 0)),
            pl.BlockSpec((1, TILE_V), lambda i: (0, i)),
        ],
        out_specs=pl.BlockSpec((BATCH, TILE_V), lambda i: (0, i)),
        out_shape=jax.ShapeDtypeStruct((BATCH, VOCAB), jnp.float32),
        scratch_shapes=[pltpu.VMEM((BATCH, DIM), jnp.bfloat16)],
        interpret=interpret,
    )(x, W, b2)


@jax.jit
def kernel(_inputs, target_table, W, b):
    idx = _inputs.astype(jnp.int32)
    x_raw = _sc_gather(target_table, idx)
    return _matmul(x_raw, W, b.reshape(1, VOCAB))
